# baseline (speedup 1.0000x reference)
import functools

import jax
import jax.numpy as jnp
from jax import lax
from jax.experimental import pallas as pl
from jax.experimental.pallas import tpu as pltpu
from jax.experimental.pallas import tpu_sc as plsc

LANES = 16
CHUNK = 128


def _mm_body(x_ref, w_ref, o_ref):
    o_ref[...] = jnp.dot(x_ref[...], w_ref[...],
                         preferred_element_type=jnp.float32)


def _matmul(x, w):
    n, d_in = x.shape
    d_out = w.shape[1]
    bm = 1000
    return pl.pallas_call(
        _mm_body,
        grid=(n // bm,),
        in_specs=[
            pl.BlockSpec((bm, d_in), lambda i: (i, 0)),
            pl.BlockSpec((d_in, d_out), lambda i: (0, 0)),
        ],
        out_specs=pl.BlockSpec((bm, d_out), lambda i: (i, 0)),
        out_shape=jax.ShapeDtypeStruct((n, d_out), jnp.float32),
    )(x, w)


def _add_body(a_ref, b_ref, o_ref):
    o_ref[...] = a_ref[...] + b_ref[...]


def _add_relu_body(a_ref, b_ref, o_ref):
    o_ref[...] = jnp.maximum(a_ref[...] + b_ref[...], 0.0)


def _combine(p0, p1, relu):
    n, d = p0.shape
    bm = 1000
    return pl.pallas_call(
        _add_relu_body if relu else _add_body,
        grid=(n // bm,),
        in_specs=[
            pl.BlockSpec((bm, d), lambda i: (i, 0)),
            pl.BlockSpec((bm, d), lambda i: (i, 0)),
        ],
        out_specs=pl.BlockSpec((bm, d), lambda i: (i, 0)),
        out_shape=jax.ShapeDtypeStruct((n, d), jnp.float32),
    )(p0, p1)


@functools.cache
def _make_hop(n, d, e_pad):
    info = plsc.get_sparse_core_info()
    nc, ns = info.num_cores, info.num_subcores
    nw = nc * ns
    epw = e_pad // nw
    nchunks = epw // CHUNK
    rstride = (n // ns) // 8 * 8
    tail = n - ns * rstride
    zrows = 128
    assert epw % CHUNK == 0 and 0 <= tail < 128 and tail % 8 == 0

    mesh = plsc.VectorSubcoreMesh(core_axis_name="c", subcore_axis_name="s")

    @functools.partial(
        pl.kernel,
        mesh=mesh,
        out_type=jax.ShapeDtypeStruct((nc, n, d), jnp.float32),
        scratch_types=[
            pltpu.VMEM_SHARED((n, d), jnp.float32),
            pltpu.VMEM((CHUNK, d), jnp.float32),
            pltpu.VMEM((CHUNK,), jnp.int32),
            pltpu.VMEM((CHUNK,), jnp.int32),
            pltpu.VMEM((CHUNK,), jnp.float32),
            pltpu.VMEM((128, 128), jnp.float32),
            pltpu.SemaphoreType.DMA,
        ],
    )
    def hop(h_hbm, idx_hbm, vals_hbm, out_hbm,
            acc, gat, colv, rowv, valv, zbuf, sem):
        cid = lax.axis_index("c")
        sid = lax.axis_index("s")
        wid = sid * nc + cid

        zero16 = jnp.zeros((LANES,), jnp.float32)

        def zb(i, carry):
            for c8 in range(d // LANES):
                zbuf[i, pl.ds(c8 * LANES, LANES)] = zero16
            return carry

        lax.fori_loop(0, zrows, zb, 0)
        z0 = sid * rstride
        nfull = rstride // zrows
        rem = rstride - nfull * zrows
        for k in range(nfull):
            pltpu.sync_copy(zbuf, acc.at[pl.ds(z0 + k * zrows, zrows)])
        if rem:
            pltpu.sync_copy(zbuf.at[pl.ds(0, rem)],
                            acc.at[pl.ds(z0 + nfull * zrows, rem)])
        if tail:
            @pl.when(sid == 0)
            def _():
                pltpu.sync_copy(zbuf.at[pl.ds(0, tail)],
                                acc.at[pl.ds(ns * rstride, tail)])
        plsc.subcore_barrier()

        base = wid * epw

        def chunk_body(ci, carry):
            off = base + ci * CHUNK
            ioff = 2 * off
            pltpu.sync_copy(idx_hbm.at[pl.ds(ioff, CHUNK)], colv)
            pltpu.sync_copy(idx_hbm.at[pl.ds(ioff + CHUNK, CHUNK)], rowv)
            pltpu.sync_copy(vals_hbm.at[pl.ds(off, CHUNK)], valv)
            pltpu.async_copy(h_hbm.at[colv], gat, sem).wait()

            def scale(j16, c2):
                vv = valv[pl.ds(j16 * LANES, LANES)]
                for i in range(LANES):
                    v = vv[i]
                    j = j16 * LANES + i
                    for c8 in range(d // LANES):
                        sl = pl.ds(c8 * LANES, LANES)
                        gat[j, sl] = gat[j, sl] * v
                return c2

            lax.fori_loop(0, CHUNK // LANES, scale, 0)
            pltpu.sync_copy(gat, acc.at[rowv], add=True)
            return carry

        lax.fori_loop(0, nchunks, chunk_body, 0)
        plsc.subcore_barrier()

        for k in range(nfull):
            pltpu.sync_copy(acc.at[pl.ds(z0 + k * zrows, zrows)],
                            out_hbm.at[cid].at[pl.ds(z0 + k * zrows, zrows)])
        if rem:
            pltpu.sync_copy(acc.at[pl.ds(z0 + nfull * zrows, rem)],
                            out_hbm.at[cid].at[pl.ds(z0 + nfull * zrows, rem)])
        if tail:
            @pl.when(sid == 0)
            def _():
                pltpu.sync_copy(acc.at[pl.ds(ns * rstride, tail)],
                                out_hbm.at[cid].at[pl.ds(ns * rstride, tail)])

    return hop


def kernel(x, edge_index, edge_vals, W):
    n, d = x.shape
    e = edge_vals.shape[0]
    rows = edge_index[0].astype(jnp.int32)
    cols = edge_index[1].astype(jnp.int32)
    vals = edge_vals.astype(jnp.float32)

    grain = 32 * CHUNK
    e_pad = ((e + grain - 1) // grain) * grain
    if e_pad != e:
        pad = e_pad - e
        rows = jnp.concatenate([rows, jnp.zeros((pad,), jnp.int32)])
        cols = jnp.concatenate([cols, jnp.zeros((pad,), jnp.int32)])
        vals = jnp.concatenate([vals, jnp.zeros((pad,), jnp.float32)])

    packed = (jnp.stack([cols, rows])
              .reshape(2, e_pad // CHUNK, CHUNK)
              .swapaxes(0, 1)
              .reshape(-1))

    hop = _make_hop(n, d, e_pad)
    h = _matmul(x, W)
    p = hop(h, packed, vals)
    h = _combine(p[0], p[1], relu=False)
    p = hop(h, packed, vals)
    return _combine(p[0], p[1], relu=True)


# speedup vs baseline: 1.0641x; 1.0641x over previous
import functools

import jax
import jax.numpy as jnp
from jax import lax
from jax.experimental import pallas as pl
from jax.experimental.pallas import tpu as pltpu
from jax.experimental.pallas import tpu_sc as plsc

LANES = 16
CHUNK = 128


def _mm_body(x_ref, w_ref, o_ref):
    o_ref[...] = jnp.dot(x_ref[...], w_ref[...],
                         preferred_element_type=jnp.float32)


def _matmul(x, w):
    n, d_in = x.shape
    d_out = w.shape[1]
    bm = 1000
    return pl.pallas_call(
        _mm_body,
        grid=(n // bm,),
        in_specs=[
            pl.BlockSpec((bm, d_in), lambda i: (i, 0)),
            pl.BlockSpec((d_in, d_out), lambda i: (0, 0)),
        ],
        out_specs=pl.BlockSpec((bm, d_out), lambda i: (i, 0)),
        out_shape=jax.ShapeDtypeStruct((n, d_out), jnp.float32),
    )(x, w)


def _add_body(a_ref, b_ref, o_ref):
    o_ref[...] = a_ref[...] + b_ref[...]


def _add_relu_body(a_ref, b_ref, o_ref):
    o_ref[...] = jnp.maximum(a_ref[...] + b_ref[...], 0.0)


def _combine(p0, p1, relu):
    n, d = p0.shape
    bm = 1000
    return pl.pallas_call(
        _add_relu_body if relu else _add_body,
        grid=(n // bm,),
        in_specs=[
            pl.BlockSpec((bm, d), lambda i: (i, 0)),
            pl.BlockSpec((bm, d), lambda i: (i, 0)),
        ],
        out_specs=pl.BlockSpec((bm, d), lambda i: (i, 0)),
        out_shape=jax.ShapeDtypeStruct((n, d), jnp.float32),
    )(p0, p1)


@functools.cache
def _make_hop(n, d, e_pad):
    info = plsc.get_sparse_core_info()
    nc, ns = info.num_cores, info.num_subcores
    nw = nc * ns
    epw = e_pad // nw
    nchunks = epw // CHUNK
    rstride = (n // ns) // 8 * 8
    tail = n - ns * rstride
    zrows = 128
    assert epw % CHUNK == 0 and 0 <= tail < 128 and tail % 8 == 0

    mesh = plsc.VectorSubcoreMesh(core_axis_name="c", subcore_axis_name="s")

    @functools.partial(
        pl.kernel,
        mesh=mesh,
        out_type=jax.ShapeDtypeStruct((nc, n, d), jnp.float32),
        scratch_types=[
            pltpu.VMEM_SHARED((n, d), jnp.float32),
            pltpu.VMEM((CHUNK, d), jnp.float32),
            pltpu.VMEM((2 * CHUNK,), jnp.int32),
            pltpu.VMEM((CHUNK,), jnp.int32),
            pltpu.VMEM((CHUNK,), jnp.float32),
            pltpu.VMEM((128, 128), jnp.float32),
            pltpu.SemaphoreType.DMA,
        ],
    )
    def hop(h_hbm, idx_hbm, vals_hbm, out_hbm,
            acc, gat, idxv, rowv, valv, zbuf, sem):
        cid = lax.axis_index("c")
        sid = lax.axis_index("s")
        wid = sid * nc + cid

        zero16 = jnp.zeros((LANES,), jnp.float32)

        def zb(i, carry):
            for c8 in range(d // LANES):
                zbuf[i, pl.ds(c8 * LANES, LANES)] = zero16
            return carry

        lax.fori_loop(0, zrows, zb, 0)
        z0 = sid * rstride
        nfull = rstride // zrows
        rem = rstride - nfull * zrows
        for k in range(nfull):
            pltpu.sync_copy(zbuf, acc.at[pl.ds(z0 + k * zrows, zrows)])
        if rem:
            pltpu.sync_copy(zbuf.at[pl.ds(0, rem)],
                            acc.at[pl.ds(z0 + nfull * zrows, rem)])
        if tail:
            @pl.when(sid == 0)
            def _():
                pltpu.sync_copy(zbuf.at[pl.ds(0, tail)],
                                acc.at[pl.ds(ns * rstride, tail)])
        plsc.subcore_barrier()

        base = wid * epw

        def chunk_body(ci, carry):
            off = base + ci * CHUNK
            ioff = 2 * off
            pltpu.sync_copy(idx_hbm.at[pl.ds(ioff, 2 * CHUNK)], idxv)
            pltpu.sync_copy(vals_hbm.at[pl.ds(off, CHUNK)], valv)
            for t in range(CHUNK // LANES):
                sl = pl.ds(t * LANES, LANES)
                rowv[sl] = idxv[pl.ds(CHUNK + t * LANES, LANES)]
            pltpu.async_copy(h_hbm.at[idxv.at[pl.ds(0, CHUNK)]], gat,
                             sem).wait()

            def scale(j16, c2):
                vv = valv[pl.ds(j16 * LANES, LANES)]
                for i in range(LANES):
                    v = vv[i]
                    j = j16 * LANES + i
                    for c8 in range(d // LANES):
                        sl = pl.ds(c8 * LANES, LANES)
                        gat[j, sl] = gat[j, sl] * v
                return c2

            lax.fori_loop(0, CHUNK // LANES, scale, 0)
            pltpu.sync_copy(gat, acc.at[rowv], add=True)
            return carry

        lax.fori_loop(0, nchunks, chunk_body, 0)
        plsc.subcore_barrier()

        for k in range(nfull):
            pltpu.sync_copy(acc.at[pl.ds(z0 + k * zrows, zrows)],
                            out_hbm.at[cid].at[pl.ds(z0 + k * zrows, zrows)])
        if rem:
            pltpu.sync_copy(acc.at[pl.ds(z0 + nfull * zrows, rem)],
                            out_hbm.at[cid].at[pl.ds(z0 + nfull * zrows, rem)])
        if tail:
            @pl.when(sid == 0)
            def _():
                pltpu.sync_copy(acc.at[pl.ds(ns * rstride, tail)],
                                out_hbm.at[cid].at[pl.ds(ns * rstride, tail)])

    return hop


def kernel(x, edge_index, edge_vals, W):
    n, d = x.shape
    e = edge_vals.shape[0]
    rows = edge_index[0].astype(jnp.int32)
    cols = edge_index[1].astype(jnp.int32)
    vals = edge_vals.astype(jnp.float32)

    grain = 32 * CHUNK
    e_pad = ((e + grain - 1) // grain) * grain
    if e_pad != e:
        pad = e_pad - e
        rows = jnp.concatenate([rows, jnp.zeros((pad,), jnp.int32)])
        cols = jnp.concatenate([cols, jnp.zeros((pad,), jnp.int32)])
        vals = jnp.concatenate([vals, jnp.zeros((pad,), jnp.float32)])

    packed = (jnp.stack([cols, rows])
              .reshape(2, e_pad // CHUNK, CHUNK)
              .swapaxes(0, 1)
              .reshape(-1))

    hop = _make_hop(n, d, e_pad)
    h = _matmul(x, W)
    p = hop(h, packed, vals)
    h = _combine(p[0], p[1], relu=False)
    p = hop(h, packed, vals)
    return _combine(p[0], p[1], relu=True)
